# baseline (speedup 1.0000x reference)
import jax
import jax.numpy as jnp
from jax.experimental import pallas as pl
from jax.experimental.pallas import tpu as pltpu

E_ = 8
TILE_T = 512
HALF_T = TILE_T // 2


def _moe_body(x1_ref, x2_ref, d1_ref, d2_ref, c1_ref, c2_ref, w_ref, b_ref,
              out_ref, xd_acc, y_s, w_s):
    p = pl.program_id(0)
    t = pl.program_id(1)
    nt = pl.num_programs(1)

    @pl.when(p == 0)
    def _dispatch_phase():
        xt1 = jnp.transpose(x1_ref[0].astype(jnp.bfloat16), (1, 0, 2))
        xt2 = jnp.transpose(x2_ref[0].astype(jnp.bfloat16), (1, 0, 2))
        dt1 = jnp.transpose(d1_ref[0].astype(jnp.bfloat16), (1, 0, 2))
        dt2 = jnp.transpose(d2_ref[0].astype(jnp.bfloat16), (1, 0, 2))

        @pl.when(t == 0)
        def _init():
            xd_acc[...] = jnp.zeros_like(xd_acc)

        chunk = w_ref[...].astype(jnp.bfloat16).reshape(2 * 768, 256)
        w_s[2 * t] = chunk[:768]
        w_s[2 * t + 1] = chunk[768:]

        for e in range(E_):
            xd_acc[e] += jax.lax.dot_general(
                dt1[e], xt1[e], (((0,), (0,)), ((), ())),
                preferred_element_type=jnp.float32,
            ) + jax.lax.dot_general(
                dt2[e], xt2[e], (((0,), (0,)), ((), ())),
                preferred_element_type=jnp.float32,
            )

        @pl.when(t == nt - 1)
        def _expert_matmul():
            for e in range(E_):
                y = jax.lax.dot_general(
                    xd_acc[e].astype(jnp.bfloat16), w_s[e],
                    (((1,), (1,)), ((), ())),
                    preferred_element_type=jnp.float32,
                )
                y_s[e] = (y + b_ref[...]).astype(jnp.bfloat16)

    @pl.when(p == 1)
    def _combine_phase():
        ct1 = jnp.transpose(c1_ref[0].astype(jnp.bfloat16), (1, 0, 2))
        ct2 = jnp.transpose(c2_ref[0].astype(jnp.bfloat16), (1, 0, 2))
        acc1 = jnp.zeros((HALF_T, 768), jnp.float32)
        acc2 = jnp.zeros((HALF_T, 768), jnp.float32)
        for e in range(E_):
            acc1 += jnp.dot(ct1[e], y_s[e], preferred_element_type=jnp.float32)
            acc2 += jnp.dot(ct2[e], y_s[e], preferred_element_type=jnp.float32)
        out_ref[:HALF_T] = acc1
        out_ref[HALF_T:] = acc2


def kernel(x, combine_array, dispatch_mask, W, b):
    B, T, E, I = x.shape
    C = combine_array.shape[-1]
    O = W.shape[0]
    nt = T // TILE_T
    b2 = b.reshape(1, O)

    def stream_half(a):
        return pl.BlockSpec(
            (1, HALF_T, E, I),
            lambda p, t, a=a: (0, jnp.where(p == 0, 2 * t + a, 2 * nt - 2 + a), 0, 0))

    def comb_half(a):
        return pl.BlockSpec(
            (1, HALF_T, E, C),
            lambda p, t, a=a: (0, jnp.where(p == 0, a, 2 * t + a), 0, 0))

    out = pl.pallas_call(
        _moe_body,
        grid=(2, nt),
        in_specs=[
            stream_half(0), stream_half(1),
            stream_half(0), stream_half(1),
            comb_half(0), comb_half(1),
            pl.BlockSpec((192, E * I),
                         lambda p, t: (jnp.where(p == 0, t, nt - 1), 0)),
            pl.BlockSpec((1, O), lambda p, t: (0, 0)),
        ],
        out_specs=pl.BlockSpec((TILE_T, O), lambda p, t: (jnp.where(p == 0, 0, t), 0)),
        out_shape=jax.ShapeDtypeStruct((T, O), jnp.float32),
        scratch_shapes=[
            pltpu.VMEM((E_, 256, 256), jnp.float32),
            pltpu.VMEM((E_, 256, 768), jnp.bfloat16),
            pltpu.VMEM((E_, 768, 256), jnp.bfloat16),
        ],
        compiler_params=pltpu.CompilerParams(
            dimension_semantics=("arbitrary", "arbitrary"),
        ),
    )(x, x, dispatch_mask, dispatch_mask, combine_array, combine_array, W, b2)
    return out.reshape(B, T, O)
